# baseline (speedup 1.0000x reference)
import functools

import jax
import jax.numpy as jnp
from jax import lax
from jax.experimental import pallas as pl
from jax.experimental.pallas import tpu as pltpu
from jax.experimental.pallas import tpu_sc as plsc

VOCAB_SIZE = 1000000
EMBED_DIM = 32
BATCH = 4096
POS = 200

HALF = POS // 2
NUM_WORKERS = 32
B_PER_W = BATCH // NUM_WORKERS
NB = 8
GROUPS = B_PER_W // NB

_mesh = plsc.VectorSubcoreMesh(core_axis_name="c", subcore_axis_name="s")


@functools.partial(
    pl.kernel,
    mesh=_mesh,
    out_type=jax.ShapeDtypeStruct((BATCH, POS, EMBED_DIM), jnp.float32),
    scratch_types=[
        pltpu.VMEM((2 * B_PER_W, HALF), jnp.int32),
        pltpu.VMEM((NB, POS, EMBED_DIM), jnp.float32),
        pltpu.SemaphoreType.DMA,
    ],
    compiler_params=pltpu.CompilerParams(use_tc_tiling_on_sc=False),
)
def _embed_gather(idx_hbm, table_hbm, out_hbm, idx_v, buf, sem):
    wid = lax.axis_index("s") * 2 + lax.axis_index("c")
    bbase = wid * B_PER_W
    pltpu.sync_copy(idx_hbm.at[pl.ds(2 * bbase, 2 * B_PER_W)], idx_v)

    def body(g, carry):
        for ib in range(NB):
            for h in range(2):
                pltpu.async_copy(
                    table_hbm.at[idx_v.at[2 * (g * NB + ib) + h]],
                    buf.at[ib, pl.ds(h * HALF, HALF)],
                    sem,
                )
        pltpu.make_async_copy(out_hbm.at[pl.ds(0, NB)], buf, sem).wait()
        pltpu.sync_copy(buf, out_hbm.at[pl.ds(bbase + g * NB, NB)])
        return carry

    lax.fori_loop(0, GROUPS, body, 0)


def kernel(x, weight):
    idx = x.reshape(2 * BATCH, HALF).astype(jnp.int32)
    return _embed_gather(idx, weight)
